# baseline (speedup 1.0000x reference)
import functools

import jax
import jax.numpy as jnp
from jax import lax
from jax.experimental import pallas as pl
from jax.experimental.pallas import tpu as pltpu
from jax.experimental.pallas import tpu_sc as plsc

N = 10000
E = 320000
C = 128
NB = 3

NC = 2
NS = 16
K = 128
EPW = E // (NC * NS)
STEPS = 80
EPW_PAD = STEPS * K
BS = 16
NBLK = STEPS // BS
ROWS_PER_TILE = 632
NP = NS * ROWS_PER_TILE


def _edge_agg_body(y_hbm, src_hbm, dst_hbm, zeros_hbm, out_hbm,
                   src_v, dst_v, rows0, rows1, acc, g0, g1, s0, s1, isem):
    c = lax.axis_index("c")
    s = lax.axis_index("s")
    r0 = s * ROWS_PER_TILE
    rows = (rows0, rows1)
    gsem = (g0, g1)
    ssem = (s0, s1)

    pltpu.async_copy(src_hbm.at[c, s, 0], src_v.at[0], isem)
    pltpu.async_copy(dst_hbm.at[c, s, 0], dst_v.at[0], isem)
    pltpu.sync_copy(zeros_hbm.at[pl.ds(r0, ROWS_PER_TILE)],
                    acc.at[pl.ds(r0, ROWS_PER_TILE)])
    plsc.subcore_barrier()

    def gather(t, buf, sem):
        blk, off = divmod(t, BS)
        pltpu.async_copy(y_hbm.at[src_v.at[blk % 2, off]], buf, sem)

    def scat(t, buf, sem):
        blk, off = divmod(t, BS)
        pltpu.async_copy(buf, acc.at[dst_v.at[blk % 2, off]], sem, add=True)

    def wait_rows(sem):
        pltpu.make_async_copy(y_hbm.at[pl.ds(0, K)], rows0, sem).wait()

    def wait_idx():
        pltpu.make_async_copy(src_hbm.at[0, 0, 0], src_v.at[0], isem).wait()

    wait_idx()
    wait_idx()
    gather(0, rows0, g0)
    for t in range(STEPS):
        b = t % 2
        wait_rows(gsem[b])
        scat(t, rows[b], ssem[b])
        if t + 1 < STEPS:
            if t > 0:
                wait_rows(ssem[1 - b])
            if t % BS == 2 and t // BS + 1 < NBLK:
                nb = t // BS + 1
                pltpu.async_copy(src_hbm.at[c, s, nb], src_v.at[nb % 2], isem)
                pltpu.async_copy(dst_hbm.at[c, s, nb], dst_v.at[nb % 2], isem)
            if (t + 1) % BS == 0:
                wait_idx()
                wait_idx()
            gather(t + 1, rows[1 - b], gsem[1 - b])
    wait_rows(s0)
    wait_rows(s1)
    plsc.subcore_barrier()
    pltpu.sync_copy(acc.at[pl.ds(r0, ROWS_PER_TILE)],
                    out_hbm.at[c, pl.ds(r0, ROWS_PER_TILE)])


_edge_agg = pl.kernel(
    _edge_agg_body,
    out_type=jax.ShapeDtypeStruct((NC, NP, C), jnp.float32),
    mesh=plsc.VectorSubcoreMesh(core_axis_name="c", subcore_axis_name="s"),
    scratch_types=[
        pltpu.VMEM((2, BS, K), jnp.int32),
        pltpu.VMEM((2, BS, K), jnp.int32),
        pltpu.VMEM((K, C), jnp.float32),
        pltpu.VMEM((K, C), jnp.float32),
        pltpu.VMEM_SHARED((NP, C), jnp.float32),
        pltpu.SemaphoreType.DMA,
        pltpu.SemaphoreType.DMA,
        pltpu.SemaphoreType.DMA,
        pltpu.SemaphoreType.DMA,
        pltpu.SemaphoreType.DMA,
    ],
)


def _mm2_body(h_ref, wn_ref, ws_ref, b_ref, y_ref, base_ref):
    h = h_ref[...]
    y_ref[...] = jnp.dot(h, wn_ref[...], preferred_element_type=jnp.float32)
    base_ref[...] = (jnp.dot(h, ws_ref[...], preferred_element_type=jnp.float32)
                     + b_ref[...])


_mm2 = pl.pallas_call(
    _mm2_body,
    out_shape=(jax.ShapeDtypeStruct((N, C), jnp.float32),
               jax.ShapeDtypeStruct((N, C), jnp.float32)),
)


def _bn_body(parts_ref, base_ref, g_ref, be_ref, idn_ref, o_ref, *, residual):
    t = parts_ref[0, :N, :] + parts_ref[1, :N, :] + base_ref[...]
    mu = jnp.mean(t, axis=0, keepdims=True)
    d = t - mu
    var = jnp.mean(d * d, axis=0, keepdims=True)
    out = d * lax.rsqrt(var + 1e-3) * g_ref[...] + be_ref[...]
    if residual:
        out = out + idn_ref[...]
    o_ref[...] = jnp.maximum(out, 0.0)


def _make_bn(residual):
    return pl.pallas_call(
        functools.partial(_bn_body, residual=residual),
        out_shape=jax.ShapeDtypeStruct((N, C), jnp.float32),
    )


_bn_plain = _make_bn(False)
_bn_res = _make_bn(True)


def kernel(x, edge_index, Wn, Ws, b, gamma, beta):
    src = edge_index[0].astype(jnp.int32)
    dst = edge_index[1].astype(jnp.int32)
    pad = EPW_PAD * NC * NS - E
    src = jnp.concatenate([src, jnp.zeros((pad,), jnp.int32)])
    dst = jnp.concatenate([dst, jnp.full((pad,), N, jnp.int32)])
    src_g = src.reshape(NC, NS, NBLK, BS, K)
    dst_g = dst.reshape(NC, NS, NBLK, BS, K)
    zeros = jnp.zeros((NP, C), jnp.float32)

    h = x
    for i in range(NB):
        identity = h
        for j in range(2):
            y, base = _mm2(h, Wn[i, j], Ws[i, j], b[i, j][None])
            parts = _edge_agg(y, src_g, dst_g, zeros)
            if j == 0:
                h = _bn_plain(parts, base, gamma[i, j][None], beta[i, j][None],
                              identity)
            else:
                h = _bn_res(parts, base, gamma[i, j][None], beta[i, j][None],
                            identity)
    return h


# speedup vs baseline: 1.0318x; 1.0318x over previous
import functools

import jax
import jax.numpy as jnp
from jax import lax
from jax.experimental import pallas as pl
from jax.experimental.pallas import tpu as pltpu
from jax.experimental.pallas import tpu_sc as plsc

N = 10000
E = 320000
C = 128
NB = 3

NC = 2
NS = 16
K = 128
EPW = E // (NC * NS)
STEPS = 80
EPW_PAD = STEPS * K
BS = 20
NBLK = STEPS // BS
ROWS_PER_TILE = 632
NP = NS * ROWS_PER_TILE


def _edge_agg_body(y_hbm, src_hbm, dst_hbm, zeros_hbm, out_hbm,
                   src_v, dst_v, rows0, rows1, acc, g0, g1, s0, s1, isem):
    c = lax.axis_index("c")
    s = lax.axis_index("s")
    r0 = s * ROWS_PER_TILE
    rows = (rows0, rows1)
    gsem = (g0, g1)
    ssem = (s0, s1)

    pltpu.async_copy(src_hbm.at[c, s, 0], src_v.at[0], isem)
    pltpu.async_copy(dst_hbm.at[c, s, 0], dst_v.at[0], isem)
    pltpu.sync_copy(zeros_hbm.at[pl.ds(r0, ROWS_PER_TILE)],
                    acc.at[pl.ds(r0, ROWS_PER_TILE)])
    plsc.subcore_barrier()

    def wait_rows(sem):
        pltpu.make_async_copy(y_hbm.at[pl.ds(0, K)], rows0, sem).wait()

    def wait_idx():
        pltpu.make_async_copy(src_hbm.at[0, 0, 0], src_v.at[0], isem).wait()

    wait_idx()
    wait_idx()

    def block(blk, carry):
        cur = lax.rem(blk, 2)
        nxt = 1 - cur
        pltpu.async_copy(src_hbm.at[c, s, blk + 1], src_v.at[nxt], isem)
        pltpu.async_copy(dst_hbm.at[c, s, blk + 1], dst_v.at[nxt], isem)
        pltpu.async_copy(y_hbm.at[src_v.at[cur, 0]], rows0, g0)
        for off in range(BS):
            b = off & 1
            wait_rows(gsem[b])
            if off + 1 < BS:
                if off >= 1:
                    wait_rows(ssem[1 - b])
                pltpu.async_copy(y_hbm.at[src_v.at[cur, off + 1]],
                                 rows[1 - b], gsem[1 - b])
            pltpu.async_copy(rows[b], acc.at[dst_v.at[cur, off]],
                             ssem[b], add=True)
        wait_rows(s0)
        wait_rows(s1)
        wait_idx()
        wait_idx()
        return carry

    lax.fori_loop(0, NBLK, block, 0)
    plsc.subcore_barrier()
    pltpu.sync_copy(acc.at[pl.ds(r0, ROWS_PER_TILE)],
                    out_hbm.at[c, pl.ds(r0, ROWS_PER_TILE)])


_edge_agg = pl.kernel(
    _edge_agg_body,
    out_type=jax.ShapeDtypeStruct((NC, NP, C), jnp.float32),
    mesh=plsc.VectorSubcoreMesh(core_axis_name="c", subcore_axis_name="s"),
    scratch_types=[
        pltpu.VMEM((2, BS, K), jnp.int32),
        pltpu.VMEM((2, BS, K), jnp.int32),
        pltpu.VMEM((K, C), jnp.float32),
        pltpu.VMEM((K, C), jnp.float32),
        pltpu.VMEM_SHARED((NP, C), jnp.float32),
        pltpu.SemaphoreType.DMA,
        pltpu.SemaphoreType.DMA,
        pltpu.SemaphoreType.DMA,
        pltpu.SemaphoreType.DMA,
        pltpu.SemaphoreType.DMA,
    ],
)


def _mm2_body(h_ref, wn_ref, ws_ref, b_ref, y_ref, base_ref):
    h = h_ref[...]
    y_ref[...] = jnp.dot(h, wn_ref[...], preferred_element_type=jnp.float32)
    base_ref[...] = (jnp.dot(h, ws_ref[...], preferred_element_type=jnp.float32)
                     + b_ref[...])


_mm2 = pl.pallas_call(
    _mm2_body,
    out_shape=(jax.ShapeDtypeStruct((N, C), jnp.float32),
               jax.ShapeDtypeStruct((N, C), jnp.float32)),
)


def _bn_body(parts_ref, base_ref, g_ref, be_ref, idn_ref, o_ref, *, residual):
    t = parts_ref[0, :N, :] + parts_ref[1, :N, :] + base_ref[...]
    mu = jnp.mean(t, axis=0, keepdims=True)
    d = t - mu
    var = jnp.mean(d * d, axis=0, keepdims=True)
    out = d * lax.rsqrt(var + 1e-3) * g_ref[...] + be_ref[...]
    if residual:
        out = out + idn_ref[...]
    o_ref[...] = jnp.maximum(out, 0.0)


def _make_bn(residual):
    return pl.pallas_call(
        functools.partial(_bn_body, residual=residual),
        out_shape=jax.ShapeDtypeStruct((N, C), jnp.float32),
    )


_bn_plain = _make_bn(False)
_bn_res = _make_bn(True)


def kernel(x, edge_index, Wn, Ws, b, gamma, beta):
    src = edge_index[0].astype(jnp.int32)
    dst = edge_index[1].astype(jnp.int32)
    pad = EPW_PAD * NC * NS - E
    src = jnp.concatenate([src, jnp.zeros((pad,), jnp.int32)])
    dst = jnp.concatenate([dst, jnp.full((pad,), N, jnp.int32)])
    junk = jnp.zeros((NC, NS, 1, BS, K), jnp.int32)
    src_g = jnp.concatenate([src.reshape(NC, NS, NBLK, BS, K), junk], axis=2)
    dst_g = jnp.concatenate([dst.reshape(NC, NS, NBLK, BS, K), junk], axis=2)
    zeros = jnp.zeros((NP, C), jnp.float32)

    h = x
    for i in range(NB):
        identity = h
        for j in range(2):
            y, base = _mm2(h, Wn[i, j], Ws[i, j], b[i, j][None])
            parts = _edge_agg(y, src_g, dst_g, zeros)
            if j == 0:
                h = _bn_plain(parts, base, gamma[i, j][None], beta[i, j][None],
                              identity)
            else:
                h = _bn_res(parts, base, gamma[i, j][None], beta[i, j][None],
                            identity)
    return h


# speedup vs baseline: 1.0370x; 1.0050x over previous
import functools

import jax
import jax.numpy as jnp
from jax import lax
from jax.experimental import pallas as pl
from jax.experimental.pallas import tpu as pltpu
from jax.experimental.pallas import tpu_sc as plsc

N = 10000
E = 320000
C = 128
NB = 3

NC = 2
NS = 16
K = 128
EPW = E // (NC * NS)
STEPS = 80
EPW_PAD = STEPS * K
BS = 20
NBLK = STEPS // BS
ROWS_PER_TILE = 632
NP = NS * ROWS_PER_TILE


def _edge_agg_body(y_hbm, src_hbm, dst_hbm, zeros_hbm, out_hbm,
                   src_v, dst_v, rows0, rows1, acc, g0, g1, s0, s1, isem):
    c = lax.axis_index("c")
    s = lax.axis_index("s")
    r0 = s * ROWS_PER_TILE
    rows = (rows0, rows1)
    gsem = (g0, g1)
    ssem = (s0, s1)

    pltpu.async_copy(src_hbm.at[c, s, 0], src_v.at[0], isem)
    pltpu.async_copy(dst_hbm.at[c, s, 0], dst_v.at[0], isem)
    pltpu.sync_copy(zeros_hbm.at[pl.ds(r0, ROWS_PER_TILE)],
                    acc.at[pl.ds(r0, ROWS_PER_TILE)])
    plsc.subcore_barrier()

    def wait_idx():
        pltpu.make_async_copy(src_hbm.at[0, 0, 0], src_v.at[0], isem).wait()

    wait_idx()
    wait_idx()

    def block(blk, carry):
        cur = lax.rem(blk, 2)
        nxt = 1 - cur
        pltpu.async_copy(src_hbm.at[c, s, blk + 1], src_v.at[nxt], isem)
        pltpu.async_copy(dst_hbm.at[c, s, blk + 1], dst_v.at[nxt], isem)
        gd = [None, None]
        sd = [None, None]
        gd[0] = pltpu.async_copy(y_hbm.at[src_v.at[cur, 0]], rows0, g0)
        for off in range(BS):
            b = off & 1
            gd[b].wait()
            if off + 1 < BS:
                if off >= 1:
                    sd[1 - b].wait()
                gd[1 - b] = pltpu.async_copy(y_hbm.at[src_v.at[cur, off + 1]],
                                             rows[1 - b], gsem[1 - b])
            sd[b] = pltpu.async_copy(rows[b], acc.at[dst_v.at[cur, off]],
                                     ssem[b], add=True)
        sd[(BS - 2) & 1].wait()
        sd[(BS - 1) & 1].wait()
        wait_idx()
        wait_idx()
        return carry

    lax.fori_loop(0, NBLK, block, 0)
    plsc.subcore_barrier()
    pltpu.sync_copy(acc.at[pl.ds(r0, ROWS_PER_TILE)],
                    out_hbm.at[c, pl.ds(r0, ROWS_PER_TILE)])


_edge_agg = pl.kernel(
    _edge_agg_body,
    out_type=jax.ShapeDtypeStruct((NC, NP, C), jnp.float32),
    mesh=plsc.VectorSubcoreMesh(core_axis_name="c", subcore_axis_name="s"),
    scratch_types=[
        pltpu.VMEM((2, BS, K), jnp.int32),
        pltpu.VMEM((2, BS, K), jnp.int32),
        pltpu.VMEM((K, C), jnp.float32),
        pltpu.VMEM((K, C), jnp.float32),
        pltpu.VMEM_SHARED((NP, C), jnp.float32),
        pltpu.SemaphoreType.DMA,
        pltpu.SemaphoreType.DMA,
        pltpu.SemaphoreType.DMA,
        pltpu.SemaphoreType.DMA,
        pltpu.SemaphoreType.DMA,
    ],
)


def _mm2_body(h_ref, wn_ref, ws_ref, b_ref, y_ref, base_ref):
    h = h_ref[...]
    y_ref[...] = jnp.dot(h, wn_ref[...], preferred_element_type=jnp.float32)
    base_ref[...] = (jnp.dot(h, ws_ref[...], preferred_element_type=jnp.float32)
                     + b_ref[...])


_mm2 = pl.pallas_call(
    _mm2_body,
    out_shape=(jax.ShapeDtypeStruct((N, C), jnp.float32),
               jax.ShapeDtypeStruct((N, C), jnp.float32)),
)


def _bn_body(parts_ref, base_ref, g_ref, be_ref, idn_ref, o_ref, *, residual):
    t = parts_ref[0, :N, :] + parts_ref[1, :N, :] + base_ref[...]
    mu = jnp.mean(t, axis=0, keepdims=True)
    d = t - mu
    var = jnp.mean(d * d, axis=0, keepdims=True)
    out = d * lax.rsqrt(var + 1e-3) * g_ref[...] + be_ref[...]
    if residual:
        out = out + idn_ref[...]
    o_ref[...] = jnp.maximum(out, 0.0)


def _make_bn(residual):
    return pl.pallas_call(
        functools.partial(_bn_body, residual=residual),
        out_shape=jax.ShapeDtypeStruct((N, C), jnp.float32),
    )


_bn_plain = _make_bn(False)
_bn_res = _make_bn(True)


def kernel(x, edge_index, Wn, Ws, b, gamma, beta):
    src = edge_index[0].astype(jnp.int32)
    dst = edge_index[1].astype(jnp.int32)
    pad = EPW_PAD * NC * NS - E
    src = jnp.concatenate([src, jnp.zeros((pad,), jnp.int32)])
    dst = jnp.concatenate([dst, jnp.full((pad,), N, jnp.int32)])
    junk = jnp.zeros((NC, NS, 1, BS, K), jnp.int32)
    src_g = jnp.concatenate([src.reshape(NC, NS, NBLK, BS, K), junk], axis=2)
    dst_g = jnp.concatenate([dst.reshape(NC, NS, NBLK, BS, K), junk], axis=2)
    zeros = jnp.zeros((NP, C), jnp.float32)

    h = x
    for i in range(NB):
        identity = h
        for j in range(2):
            y, base = _mm2(h, Wn[i, j], Ws[i, j], b[i, j][None])
            parts = _edge_agg(y, src_g, dst_g, zeros)
            if j == 0:
                h = _bn_plain(parts, base, gamma[i, j][None], beta[i, j][None],
                              identity)
            else:
                h = _bn_res(parts, base, gamma[i, j][None], beta[i, j][None],
                            identity)
    return h
